# initial kernel scaffold (unmeasured)
import jax
import jax.numpy as jnp
from jax import lax
from jax.experimental import pallas as pl
from jax.experimental.pallas import tpu as pltpu

NZ = 4
B = 4
S = 2048
HD = 2048
N = 8192
SC = S // NZ
CH_ROWS = B * SC
NT = 1024
TR = 256


def _matmul_body(x_ref, w_ref, o_ref):
    o_ref[...] = jnp.dot(
        x_ref[0], w_ref[...], preferred_element_type=jnp.float32
    )


def _compute_partials(Xb, Wb):
    grid = (NZ, N // NT, B)
    return pl.pallas_call(
        _matmul_body,
        grid=grid,
        in_specs=[
            pl.BlockSpec((1, SC, HD), lambda sc, n, b: (b, sc, 0)),
            pl.BlockSpec((HD, NT), lambda sc, n, b: (0, n)),
        ],
        out_specs=pl.BlockSpec((1, SC, NT), lambda sc, n, b: (sc, b, n)),
        out_shape=jax.ShapeDtypeStruct((NZ, CH_ROWS, N), jnp.float32),
    )(Xb, Wb)


def _rs_body(
    p_ref, out_ref, recv_ref, acc_ref,
    send_sems, recv_sems, cp_sems, a_tile, b_tile, o_tile,
):
    x = lax.axis_index("x")
    y = lax.axis_index("y")
    z = lax.axis_index("z")
    nxt = lax.rem(z + 1, NZ)
    prv = lax.rem(z + NZ - 1, NZ)

    barrier = pltpu.get_barrier_semaphore()
    pl.semaphore_signal(
        barrier, inc=1, device_id=(x, y, nxt),
        device_id_type=pl.DeviceIdType.MESH,
    )
    pl.semaphore_signal(
        barrier, inc=1, device_id=(x, y, prv),
        device_id_type=pl.DeviceIdType.MESH,
    )
    pl.semaphore_wait(barrier, 2)

    def make_rdma(src, slot):
        return pltpu.make_async_remote_copy(
            src_ref=src,
            dst_ref=recv_ref.at[slot],
            send_sem=send_sems.at[slot],
            recv_sem=recv_sems.at[slot],
            device_id=(x, y, nxt),
            device_id_type=pl.DeviceIdType.MESH,
        )

    def accumulate(a_hbm, b_hbm, dst_hbm):
        for i in range(CH_ROWS // TR):
            sl = pl.ds(i * TR, TR)
            ca = pltpu.make_async_copy(a_hbm.at[sl], a_tile, cp_sems.at[0])
            cb = pltpu.make_async_copy(b_hbm.at[sl], b_tile, cp_sems.at[1])
            ca.start()
            cb.start()
            ca.wait()
            cb.wait()
            o_tile[...] = a_tile[...] + b_tile[...]
            co = pltpu.make_async_copy(o_tile, dst_hbm.at[sl], cp_sems.at[2])
            co.start()
            co.wait()

    rdmas = [make_rdma(p_ref.at[prv], 0)]
    rdmas[0].start()

    for t in (1, 2):
        make_rdma(p_ref.at[0], t - 1).wait_recv()
        c = lax.rem(z + 2 * NZ - 1 - t, NZ)
        accumulate(recv_ref.at[t - 1], p_ref.at[c], acc_ref.at[t - 1])
        rdmas.append(make_rdma(acc_ref.at[t - 1], t))
        rdmas[-1].start()

    make_rdma(p_ref.at[0], 2).wait_recv()
    accumulate(recv_ref.at[2], p_ref.at[z], out_ref)

    for r in rdmas:
        r.wait_send()


def _reduce_scatter(P):
    out, _, _ = pl.pallas_call(
        _rs_body,
        out_shape=[
            jax.ShapeDtypeStruct((CH_ROWS, N), jnp.float32),
            jax.ShapeDtypeStruct((3, CH_ROWS, N), jnp.float32),
            jax.ShapeDtypeStruct((2, CH_ROWS, N), jnp.float32),
        ],
        in_specs=[pl.BlockSpec(memory_space=pltpu.ANY)],
        out_specs=[pl.BlockSpec(memory_space=pltpu.ANY)] * 3,
        scratch_shapes=[
            pltpu.SemaphoreType.DMA((3,)),
            pltpu.SemaphoreType.DMA((3,)),
            pltpu.SemaphoreType.DMA((3,)),
            pltpu.VMEM((TR, N), jnp.float32),
            pltpu.VMEM((TR, N), jnp.float32),
            pltpu.VMEM((TR, N), jnp.float32),
        ],
        compiler_params=pltpu.CompilerParams(collective_id=0),
    )(P)
    return out


def kernel(O, Wo):
    X = O.reshape(B, S, HD).astype(jnp.bfloat16)
    Wb = Wo.astype(jnp.bfloat16)
    P = _compute_partials(X, Wb)
    out = _reduce_scatter(P)
    return out.reshape(B, SC, N)


# baseline (device time: 2928324 ns/iter reference)
import jax
import jax.numpy as jnp
from jax import lax
from jax.experimental import pallas as pl
from jax.experimental.pallas import tpu as pltpu

NZ = 4
B = 4
S = 2048
HD = 2048
N = 8192
SC = S // NZ
CH_ROWS = B * SC
NT = 1024
TR = 256


def _matmul_body(x_ref, w_ref, o_ref):
    o_ref[0] = jnp.dot(
        x_ref[0], w_ref[...], preferred_element_type=jnp.float32
    )


def _compute_partials(Xb, Wb):
    grid = (NZ, N // NT, B)
    return pl.pallas_call(
        _matmul_body,
        grid=grid,
        in_specs=[
            pl.BlockSpec((1, SC, HD), lambda sc, n, b: (b, sc, 0)),
            pl.BlockSpec((HD, NT), lambda sc, n, b: (0, n)),
        ],
        out_specs=pl.BlockSpec((1, SC, NT), lambda sc, n, b: (sc, b, n)),
        out_shape=jax.ShapeDtypeStruct((NZ, CH_ROWS, N), jnp.float32),
    )(Xb, Wb)


def _rs_body(
    p_ref, out_ref, recv_ref, acc_ref,
    send_sems, recv_sems, cp_sems, a_tile, b_tile, o_tile,
):
    x = lax.axis_index("x")
    y = lax.axis_index("y")
    z = lax.axis_index("z")
    nxt = lax.rem(z + 1, NZ)
    prv = lax.rem(z + NZ - 1, NZ)

    barrier = pltpu.get_barrier_semaphore()
    pl.semaphore_signal(
        barrier, inc=1, device_id=(x, y, nxt),
        device_id_type=pl.DeviceIdType.MESH,
    )
    pl.semaphore_signal(
        barrier, inc=1, device_id=(x, y, prv),
        device_id_type=pl.DeviceIdType.MESH,
    )
    pl.semaphore_wait(barrier, 2)

    def make_rdma(src, slot):
        return pltpu.make_async_remote_copy(
            src_ref=src,
            dst_ref=recv_ref.at[slot],
            send_sem=send_sems.at[slot],
            recv_sem=recv_sems.at[slot],
            device_id=(x, y, nxt),
            device_id_type=pl.DeviceIdType.MESH,
        )

    def accumulate(a_hbm, b_hbm, dst_hbm):
        for i in range(CH_ROWS // TR):
            sl = pl.ds(i * TR, TR)
            ca = pltpu.make_async_copy(a_hbm.at[sl], a_tile, cp_sems.at[0])
            cb = pltpu.make_async_copy(b_hbm.at[sl], b_tile, cp_sems.at[1])
            ca.start()
            cb.start()
            ca.wait()
            cb.wait()
            o_tile[...] = a_tile[...] + b_tile[...]
            co = pltpu.make_async_copy(o_tile, dst_hbm.at[sl], cp_sems.at[2])
            co.start()
            co.wait()

    rdmas = [make_rdma(p_ref.at[prv], 0)]
    rdmas[0].start()

    for t in (1, 2):
        make_rdma(p_ref.at[0], t - 1).wait_recv()
        c = lax.rem(z + 2 * NZ - 1 - t, NZ)
        accumulate(recv_ref.at[t - 1], p_ref.at[c], acc_ref.at[t - 1])
        rdmas.append(make_rdma(acc_ref.at[t - 1], t))
        rdmas[-1].start()

    make_rdma(p_ref.at[0], 2).wait_recv()
    accumulate(recv_ref.at[2], p_ref.at[z], out_ref)

    for r in rdmas:
        r.wait_send()


def _reduce_scatter(P):
    out, _, _ = pl.pallas_call(
        _rs_body,
        out_shape=[
            jax.ShapeDtypeStruct((CH_ROWS, N), jnp.float32),
            jax.ShapeDtypeStruct((3, CH_ROWS, N), jnp.float32),
            jax.ShapeDtypeStruct((2, CH_ROWS, N), jnp.float32),
        ],
        in_specs=[pl.BlockSpec(memory_space=pltpu.MemorySpace.HBM)],
        out_specs=[pl.BlockSpec(memory_space=pltpu.MemorySpace.HBM)] * 3,
        scratch_shapes=[
            pltpu.SemaphoreType.DMA((3,)),
            pltpu.SemaphoreType.DMA((3,)),
            pltpu.SemaphoreType.DMA((3,)),
            pltpu.VMEM((TR, N), jnp.float32),
            pltpu.VMEM((TR, N), jnp.float32),
            pltpu.VMEM((TR, N), jnp.float32),
        ],
        compiler_params=pltpu.CompilerParams(collective_id=0),
    )(P)
    return out


def kernel(O, Wo):
    X = O.reshape(B, S, HD).astype(jnp.bfloat16)
    Wb = Wo.astype(jnp.bfloat16)
    P = _compute_partials(X, Wb)
    out = _reduce_scatter(P)
    return out.reshape(B, SC, N)


# device time: 1758192 ns/iter; 1.6655x vs baseline; 1.6655x over previous
import jax
import jax.numpy as jnp
from jax import lax
from jax.experimental import pallas as pl
from jax.experimental.pallas import tpu as pltpu

NZ = 4
B = 4
S = 2048
HD = 2048
N = 8192
SC = S // NZ
CH_ROWS = B * SC
NT = 1024
TR = 256


def _matmul_body(x_ref, w_ref, o_ref):
    o_ref[0] = jnp.dot(
        x_ref[0], w_ref[...], preferred_element_type=jnp.float32
    ).astype(jnp.bfloat16)


def _compute_partials(Xb, Wb):
    grid = (NZ, N // NT, B)
    return pl.pallas_call(
        _matmul_body,
        grid=grid,
        in_specs=[
            pl.BlockSpec((1, SC, HD), lambda sc, n, b: (b, sc, 0)),
            pl.BlockSpec((HD, NT), lambda sc, n, b: (0, n)),
        ],
        out_specs=pl.BlockSpec((1, SC, NT), lambda sc, n, b: (sc, b, n)),
        out_shape=jax.ShapeDtypeStruct((NZ, CH_ROWS, N), jnp.bfloat16),
    )(Xb, Wb)


def _rs_body(
    p_ref, out_ref, recv_ref, acc_ref,
    send_sems, recv_sems, cp_sems, a_tile, b_tile, o_tile, of_tile,
):
    x = lax.axis_index("x")
    y = lax.axis_index("y")
    z = lax.axis_index("z")
    nxt = lax.rem(z + 1, NZ)
    prv = lax.rem(z + NZ - 1, NZ)

    barrier = pltpu.get_barrier_semaphore()
    pl.semaphore_signal(
        barrier, inc=1, device_id=(x, y, nxt),
        device_id_type=pl.DeviceIdType.MESH,
    )
    pl.semaphore_signal(
        barrier, inc=1, device_id=(x, y, prv),
        device_id_type=pl.DeviceIdType.MESH,
    )
    pl.semaphore_wait(barrier, 2)

    def make_rdma(src, slot):
        return pltpu.make_async_remote_copy(
            src_ref=src,
            dst_ref=recv_ref.at[slot],
            send_sem=send_sems.at[slot],
            recv_sem=recv_sems.at[slot],
            device_id=(x, y, nxt),
            device_id_type=pl.DeviceIdType.MESH,
        )

    def accumulate(a_hbm, b_hbm, dst_hbm, final=False):
        dst_tile = of_tile if final else o_tile
        for i in range(CH_ROWS // TR):
            sl = pl.ds(i * TR, TR)
            ca = pltpu.make_async_copy(a_hbm.at[sl], a_tile, cp_sems.at[0])
            cb = pltpu.make_async_copy(b_hbm.at[sl], b_tile, cp_sems.at[1])
            ca.start()
            cb.start()
            ca.wait()
            cb.wait()
            acc = a_tile[...].astype(jnp.float32) + b_tile[...].astype(
                jnp.float32
            )
            dst_tile[...] = acc.astype(dst_tile.dtype)
            co = pltpu.make_async_copy(dst_tile, dst_hbm.at[sl], cp_sems.at[2])
            co.start()
            co.wait()

    rdmas = [make_rdma(p_ref.at[prv], 0)]
    rdmas[0].start()

    for t in (1, 2):
        make_rdma(p_ref.at[0], t - 1).wait_recv()
        c = lax.rem(z + 2 * NZ - 1 - t, NZ)
        accumulate(recv_ref.at[t - 1], p_ref.at[c], acc_ref.at[t - 1])
        rdmas.append(make_rdma(acc_ref.at[t - 1], t))
        rdmas[-1].start()

    make_rdma(p_ref.at[0], 2).wait_recv()
    accumulate(recv_ref.at[2], p_ref.at[z], out_ref, final=True)

    for r in rdmas:
        r.wait_send()


def _reduce_scatter(P):
    out, _, _ = pl.pallas_call(
        _rs_body,
        out_shape=[
            jax.ShapeDtypeStruct((CH_ROWS, N), jnp.float32),
            jax.ShapeDtypeStruct((3, CH_ROWS, N), jnp.bfloat16),
            jax.ShapeDtypeStruct((2, CH_ROWS, N), jnp.bfloat16),
        ],
        in_specs=[pl.BlockSpec(memory_space=pltpu.MemorySpace.HBM)],
        out_specs=[pl.BlockSpec(memory_space=pltpu.MemorySpace.HBM)] * 3,
        scratch_shapes=[
            pltpu.SemaphoreType.DMA((3,)),
            pltpu.SemaphoreType.DMA((3,)),
            pltpu.SemaphoreType.DMA((3,)),
            pltpu.VMEM((TR, N), jnp.bfloat16),
            pltpu.VMEM((TR, N), jnp.bfloat16),
            pltpu.VMEM((TR, N), jnp.bfloat16),
            pltpu.VMEM((TR, N), jnp.float32),
        ],
        compiler_params=pltpu.CompilerParams(collective_id=0),
    )(P)
    return out


def kernel(O, Wo):
    X = O.reshape(B, S, HD).astype(jnp.bfloat16)
    Wb = Wo.astype(jnp.bfloat16)
    P = _compute_partials(X, Wb)
    out = _reduce_scatter(P)
    return out.reshape(B, SC, N)
